# Initial kernel scaffold; baseline (speedup 1.0000x reference)
import jax
import jax.numpy as jnp
from jax.experimental import pallas as pl


def kernel(inputs, positive):
    raise NotImplementedError("write your pallas kernel here")


# speedup vs baseline: 57.3615x; 57.3615x over previous
import jax
import jax.numpy as jnp
from jax.experimental import pallas as pl
from jax.experimental.pallas import tpu as pltpu

_N = 4096
_D = 128
_BLK = 256
_MARGIN = 0.3


def _triphard_block(x_blk_ref, p_blk_ref, x_all_ref, out_ref):
    i = pl.program_id(0)
    x_blk = x_blk_ref[...]
    x_all = x_all_ref[...]
    g = jax.lax.dot_general(
        x_blk, x_all, (((1,), (1,)), ((), ())),
        preferred_element_type=jnp.float32)
    sq_blk = jnp.sum(x_blk * x_blk, axis=1, keepdims=True)
    sq_all = jnp.sum(x_all * x_all, axis=1, keepdims=True)
    d2 = sq_blk + sq_all.T - 2.0 * g

    col = jax.lax.broadcasted_iota(jnp.int32, d2.shape, 1)
    i1 = jnp.argmin(d2, axis=1)[:, None]
    d2 = jnp.where(col == i1, jnp.inf, d2)
    i2 = jnp.argmin(d2, axis=1)[:, None]
    d2 = jnp.where(col == i2, jnp.inf, d2)
    v3 = jnp.min(d2, axis=1, keepdims=True)
    d_an = jnp.sqrt(jnp.clip(v3, 1e-12, None))

    diff = x_blk - p_blk_ref[...] + 1e-6
    d_ap = jnp.sqrt(jnp.sum(diff * diff, axis=1, keepdims=True))
    part = jnp.sum(jnp.maximum(d_ap - d_an + _MARGIN, 0.0),
                   keepdims=True)

    @pl.when(i == 0)
    def _init():
        out_ref[...] = jnp.zeros((1, 1), jnp.float32)
    out_ref[...] += part
    @pl.when(i == _N // _BLK - 1)
    def _finish():
        out_ref[...] = out_ref[...] * (1.0 / _N)


def kernel(inputs, positive):
    grid = (_N // _BLK,)
    out = pl.pallas_call(
        _triphard_block,
        grid=grid,
        in_specs=[
            pl.BlockSpec((_BLK, _D), lambda i: (i, 0)),
            pl.BlockSpec((_BLK, _D), lambda i: (i, 0)),
            pl.BlockSpec((_N, _D), lambda i: (0, 0)),
        ],
        out_specs=pl.BlockSpec((1, 1), lambda i: (0, 0)),
        out_shape=jax.ShapeDtypeStruct((1, 1), jnp.float32),
    )(inputs, positive, inputs)
    return out[0, 0]


# speedup vs baseline: 151.7424x; 2.6454x over previous
import jax
import jax.numpy as jnp
from jax.experimental import pallas as pl
from jax.experimental.pallas import tpu as pltpu

_N = 4096
_D = 128
_BLK = 256
_MARGIN = 0.3


def _triphard_block(x_blk_ref, p_blk_ref, x_all_ref, out_ref, sq_ref):
    i = pl.program_id(0)
    x_all = x_all_ref[...]

    @pl.when(i == 0)
    def _norms():
        x2 = x_all * x_all
        ones = jnp.ones((8, _D), jnp.float32)
        sq_ref[...] = jax.lax.dot_general(
            ones, x2, (((1,), (1,)), ((), ())),
            preferred_element_type=jnp.float32)

    x_blk = x_blk_ref[...]
    g = jax.lax.dot_general(
        -2.0 * x_blk, x_all, (((1,), (1,)), ((), ())),
        preferred_element_type=jnp.float32)
    t = sq_ref[0:1, :] + g

    col = jax.lax.broadcasted_iota(jnp.int32, t.shape, 1)
    row = jax.lax.broadcasted_iota(jnp.int32, t.shape, 0) + i * _BLK
    t = jnp.where(col == row, jnp.inf, t)
    m1 = jnp.min(t, axis=1, keepdims=True)
    t = jnp.where(t == m1, jnp.inf, t)
    m2 = jnp.min(t, axis=1, keepdims=True)
    sq_blk = jnp.sum(x_blk * x_blk, axis=1, keepdims=True)
    d_an = jnp.sqrt(jnp.clip(m2 + sq_blk, 1e-12, None))

    diff = x_blk - p_blk_ref[...] + 1e-6
    d_ap = jnp.sqrt(jnp.sum(diff * diff, axis=1, keepdims=True))
    part = jnp.sum(jnp.maximum(d_ap - d_an + _MARGIN, 0.0),
                   keepdims=True)

    @pl.when(i == 0)
    def _init():
        out_ref[...] = jnp.zeros((1, 1), jnp.float32)
    out_ref[...] += part

    @pl.when(i == _N // _BLK - 1)
    def _finish():
        out_ref[...] = out_ref[...] * (1.0 / _N)


def kernel(inputs, positive):
    grid = (_N // _BLK,)
    out = pl.pallas_call(
        _triphard_block,
        grid=grid,
        in_specs=[
            pl.BlockSpec((_BLK, _D), lambda i: (i, 0)),
            pl.BlockSpec((_BLK, _D), lambda i: (i, 0)),
            pl.BlockSpec((_N, _D), lambda i: (0, 0)),
        ],
        out_specs=pl.BlockSpec((1, 1), lambda i: (0, 0)),
        out_shape=jax.ShapeDtypeStruct((1, 1), jnp.float32),
        scratch_shapes=[pltpu.VMEM((8, _N), jnp.float32)],
    )(inputs, positive, inputs)
    return out[0, 0]


# speedup vs baseline: 216.4972x; 1.4267x over previous
import jax
import jax.numpy as jnp
from jax.experimental import pallas as pl
from jax.experimental.pallas import tpu as pltpu

_N = 4096
_D = 128
_BLK = 256
_SUB = 64
_MARGIN = 0.3


def _triphard_block(x_blk_ref, p_blk_ref, x_all_ref, out_ref, sq_ref,
                    xb_ref):
    i = pl.program_id(0)

    @pl.when(i == 0)
    def _norms():
        x_all = x_all_ref[...]
        x2 = x_all * x_all
        ones = jnp.ones((8, _D), jnp.float32)
        sq_ref[...] = jax.lax.dot_general(
            ones, x2, (((1,), (1,)), ((), ())),
            preferred_element_type=jnp.float32)
        xb_ref[...] = x_all.astype(jnp.bfloat16)

    x_blk = x_blk_ref[...]
    g = jax.lax.dot_general(
        (-2.0 * x_blk).astype(jnp.bfloat16), xb_ref[...],
        (((1,), (1,)), ((), ())),
        preferred_element_type=jnp.float32)
    sq_row = sq_ref[0:1, :]

    inf = jnp.float32(jnp.inf)
    d_an_subs = []
    for rb in range(_BLK // _SUB):
        m1 = jnp.full((_SUB, _D), inf)
        m2 = jnp.full((_SUB, _D), inf)
        for k in range(_N // _D):
            v = sq_row[:, k * _D:(k + 1) * _D] + \
                g[rb * _SUB:(rb + 1) * _SUB, k * _D:(k + 1) * _D]
            hi1 = jnp.maximum(m1, v)
            m1 = jnp.minimum(m1, v)
            m2 = jnp.minimum(m2, hi1)
        lane = jax.lax.broadcasted_iota(jnp.int32, (_SUB, _D), 1)
        row = jax.lax.broadcasted_iota(jnp.int32, (_SUB, _D), 0)
        m1 = jnp.where(lane == (row + rb * _SUB) % _D, inf, m1)
        cand = jnp.concatenate([m1, m2], axis=1)
        mA = jnp.min(cand, axis=1, keepdims=True)
        cand = jnp.where(cand == mA, inf, cand)
        mB = jnp.min(cand, axis=1, keepdims=True)
        d_an_subs.append(mB)
    m_rank2 = jnp.concatenate(d_an_subs, axis=0)

    sq_blk = jnp.sum(x_blk * x_blk, axis=1, keepdims=True)
    d_an = jnp.sqrt(jnp.clip(m_rank2 + sq_blk, 1e-12, None))

    diff = x_blk - p_blk_ref[...] + 1e-6
    d_ap = jnp.sqrt(jnp.sum(diff * diff, axis=1, keepdims=True))
    part = jnp.sum(jnp.maximum(d_ap - d_an + _MARGIN, 0.0),
                   keepdims=True)

    @pl.when(i == 0)
    def _init():
        out_ref[...] = jnp.zeros((1, 1), jnp.float32)
    out_ref[...] += part

    @pl.when(i == _N // _BLK - 1)
    def _finish():
        out_ref[...] = out_ref[...] * (1.0 / _N)


def kernel(inputs, positive):
    grid = (_N // _BLK,)
    out = pl.pallas_call(
        _triphard_block,
        grid=grid,
        in_specs=[
            pl.BlockSpec((_BLK, _D), lambda i: (i, 0)),
            pl.BlockSpec((_BLK, _D), lambda i: (i, 0)),
            pl.BlockSpec((_N, _D), lambda i: (0, 0)),
        ],
        out_specs=pl.BlockSpec((1, 1), lambda i: (0, 0)),
        out_shape=jax.ShapeDtypeStruct((1, 1), jnp.float32),
        scratch_shapes=[pltpu.VMEM((8, _N), jnp.float32),
                        pltpu.VMEM((_N, _D), jnp.bfloat16)],
    )(inputs, positive, inputs)
    return out[0, 0]


# speedup vs baseline: 282.2470x; 1.3037x over previous
import jax
import jax.numpy as jnp
from jax.experimental import pallas as pl
from jax.experimental.pallas import tpu as pltpu

_N = 4096
_D = 128
_BLK = 2048
_SUB = 64
_MARGIN = 0.3


def _triphard_block(x_blk_ref, p_blk_ref, x_all_ref, out_ref, sq_ref,
                    xb_ref):
    i = pl.program_id(0)

    @pl.when(i == 0)
    def _norms():
        x_all = x_all_ref[...]
        x2 = x_all * x_all
        ones = jnp.ones((8, _D), jnp.float32)
        sq_ref[...] = jax.lax.dot_general(
            ones, x2, (((1,), (1,)), ((), ())),
            preferred_element_type=jnp.float32)
        xb_ref[...] = x_all.astype(jnp.bfloat16)

    x_blk = x_blk_ref[...]
    g = jax.lax.dot_general(
        (-2.0 * x_blk).astype(jnp.bfloat16), xb_ref[...],
        (((1,), (1,)), ((), ())),
        preferred_element_type=jnp.float32)
    sq_row = sq_ref[0:1, :]

    inf = jnp.float32(jnp.inf)
    d_an_subs = []
    for rb in range(_BLK // _SUB):
        m1 = jnp.full((_SUB, _D), inf)
        m2 = jnp.full((_SUB, _D), inf)
        for k in range(_N // _D):
            v = sq_row[:, k * _D:(k + 1) * _D] + \
                g[rb * _SUB:(rb + 1) * _SUB, k * _D:(k + 1) * _D]
            hi1 = jnp.maximum(m1, v)
            m1 = jnp.minimum(m1, v)
            m2 = jnp.minimum(m2, hi1)
        lane = jax.lax.broadcasted_iota(jnp.int32, (_SUB, _D), 1)
        row = jax.lax.broadcasted_iota(jnp.int32, (_SUB, _D), 0)
        m1 = jnp.where(lane == (row + rb * _SUB) % _D, inf, m1)
        cand = jnp.concatenate([m1, m2], axis=1)
        mA = jnp.min(cand, axis=1, keepdims=True)
        cand = jnp.where(cand == mA, inf, cand)
        mB = jnp.min(cand, axis=1, keepdims=True)
        d_an_subs.append(mB)
    m_rank2 = jnp.concatenate(d_an_subs, axis=0)

    sq_blk = jnp.sum(x_blk * x_blk, axis=1, keepdims=True)
    d_an = jnp.sqrt(jnp.clip(m_rank2 + sq_blk, 1e-12, None))

    diff = x_blk - p_blk_ref[...] + 1e-6
    d_ap = jnp.sqrt(jnp.sum(diff * diff, axis=1, keepdims=True))
    part = jnp.sum(jnp.maximum(d_ap - d_an + _MARGIN, 0.0),
                   keepdims=True)

    @pl.when(i == 0)
    def _init():
        out_ref[...] = jnp.zeros((1, 1), jnp.float32)
    out_ref[...] += part

    @pl.when(i == _N // _BLK - 1)
    def _finish():
        out_ref[...] = out_ref[...] * (1.0 / _N)


def kernel(inputs, positive):
    grid = (_N // _BLK,)
    out = pl.pallas_call(
        _triphard_block,
        grid=grid,
        in_specs=[
            pl.BlockSpec((_BLK, _D), lambda i: (i, 0)),
            pl.BlockSpec((_BLK, _D), lambda i: (i, 0)),
            pl.BlockSpec((_N, _D), lambda i: (0, 0)),
        ],
        out_specs=pl.BlockSpec((1, 1), lambda i: (0, 0)),
        out_shape=jax.ShapeDtypeStruct((1, 1), jnp.float32),
        scratch_shapes=[pltpu.VMEM((8, _N), jnp.float32),
                        pltpu.VMEM((_N, _D), jnp.bfloat16)],
    )(inputs, positive, inputs)
    return out[0, 0]


# speedup vs baseline: 285.5714x; 1.0118x over previous
import jax
import jax.numpy as jnp
from jax.experimental import pallas as pl
from jax.experimental.pallas import tpu as pltpu

_N = 4096
_D = 128
_BLK = 1024
_SUB = 64
_MARGIN = 0.3


def _triphard_block(p_blk_ref, x_all_ref, out_ref, sq_ref, xb_ref):
    i = pl.program_id(0)

    @pl.when(i == 0)
    def _norms():
        x_all = x_all_ref[...]
        x2 = x_all * x_all
        ones = jnp.ones((8, _D), jnp.float32)
        sq_ref[...] = jax.lax.dot_general(
            ones, x2, (((1,), (1,)), ((), ())),
            preferred_element_type=jnp.float32)
        xb_ref[...] = x_all.astype(jnp.bfloat16)

    x_blk = x_all_ref[pl.ds(i * _BLK, _BLK), :]
    g = jax.lax.dot_general(
        (-2.0 * x_blk).astype(jnp.bfloat16), xb_ref[...],
        (((1,), (1,)), ((), ())),
        preferred_element_type=jnp.float32)
    sq_row = sq_ref[0:1, :]

    inf = jnp.float32(jnp.inf)
    d_an_subs = []
    for rb in range(_BLK // _SUB):
        m1 = jnp.full((_SUB, _D), inf)
        m2 = jnp.full((_SUB, _D), inf)
        for k in range(_N // _D):
            v = sq_row[:, k * _D:(k + 1) * _D] + \
                g[rb * _SUB:(rb + 1) * _SUB, k * _D:(k + 1) * _D]
            hi1 = jnp.maximum(m1, v)
            m1 = jnp.minimum(m1, v)
            m2 = jnp.minimum(m2, hi1)
        lane = jax.lax.broadcasted_iota(jnp.int32, (_SUB, _D), 1)
        row = jax.lax.broadcasted_iota(jnp.int32, (_SUB, _D), 0)
        m1 = jnp.where(lane == (row + rb * _SUB) % _D, inf, m1)
        cand = jnp.concatenate([m1, m2], axis=1)
        mA = jnp.min(cand, axis=1, keepdims=True)
        cand = jnp.where(cand == mA, inf, cand)
        mB = jnp.min(cand, axis=1, keepdims=True)
        d_an_subs.append(mB)
    m_rank2 = jnp.concatenate(d_an_subs, axis=0)

    sq_blk = jnp.sum(x_blk * x_blk, axis=1, keepdims=True)
    d_an = jnp.sqrt(jnp.clip(m_rank2 + sq_blk, 1e-12, None))

    diff = x_blk - p_blk_ref[...] + 1e-6
    d_ap = jnp.sqrt(jnp.sum(diff * diff, axis=1, keepdims=True))
    part = jnp.sum(jnp.maximum(d_ap - d_an + _MARGIN, 0.0),
                   keepdims=True)

    @pl.when(i == 0)
    def _init():
        out_ref[...] = jnp.zeros((1, 1), jnp.float32)
    out_ref[...] += part

    @pl.when(i == _N // _BLK - 1)
    def _finish():
        out_ref[...] = out_ref[...] * (1.0 / _N)


def kernel(inputs, positive):
    grid = (_N // _BLK,)
    out = pl.pallas_call(
        _triphard_block,
        grid=grid,
        in_specs=[
            pl.BlockSpec((_BLK, _D), lambda i: (i, 0)),
            pl.BlockSpec((_N, _D), lambda i: (0, 0)),
        ],
        out_specs=pl.BlockSpec((1, 1), lambda i: (0, 0)),
        out_shape=jax.ShapeDtypeStruct((1, 1), jnp.float32),
        scratch_shapes=[pltpu.VMEM((8, _N), jnp.float32),
                        pltpu.VMEM((_N, _D), jnp.bfloat16)],
    )(positive, inputs)
    return out[0, 0]


# speedup vs baseline: 311.4758x; 1.0907x over previous
import jax
import jax.numpy as jnp
from jax.experimental import pallas as pl
from jax.experimental.pallas import tpu as pltpu

_N = 4096
_D = 128
_BLK = 1024
_SUB = 64
_MARGIN = 0.3


def _triphard_block(p_blk_ref, x_all_ref, out_ref, sq_ref, xb_ref):
    i = pl.program_id(0)

    @pl.when(i == 0)
    def _norms():
        x_all = x_all_ref[...]
        x2 = x_all * x_all
        ones = jnp.ones((8, _D), jnp.float32)
        sq_ref[...] = jax.lax.dot_general(
            ones, x2, (((1,), (1,)), ((), ())),
            preferred_element_type=jnp.float32).astype(jnp.bfloat16)
        xb_ref[...] = x_all.astype(jnp.bfloat16)

    x_blk = x_all_ref[pl.ds(i * _BLK, _BLK), :]
    g = jax.lax.dot_general(
        (-2.0 * x_blk).astype(jnp.bfloat16), xb_ref[...],
        (((1,), (1,)), ((), ())),
        preferred_element_type=jnp.float32).astype(jnp.bfloat16)
    sq_row = sq_ref[0:1, :]

    inf = jnp.bfloat16(jnp.inf)
    d_an_subs = []
    for rb in range(_BLK // _SUB):
        m1 = jnp.full((_SUB, _D), inf, jnp.bfloat16)
        m2 = jnp.full((_SUB, _D), inf, jnp.bfloat16)
        for k in range(_N // _D):
            v = sq_row[:, k * _D:(k + 1) * _D] + \
                g[rb * _SUB:(rb + 1) * _SUB, k * _D:(k + 1) * _D]
            hi1 = jnp.maximum(m1, v)
            m1 = jnp.minimum(m1, v)
            m2 = jnp.minimum(m2, hi1)
        lane = jax.lax.broadcasted_iota(jnp.int32, (_SUB, _D), 1)
        row = jax.lax.broadcasted_iota(jnp.int32, (_SUB, _D), 0)
        m1 = jnp.where(lane == (row + rb * _SUB) % _D, inf, m1)
        cand = jnp.concatenate([m1, m2], axis=1)
        mA = jnp.min(cand, axis=1, keepdims=True)
        cand = jnp.where(cand == mA, inf, cand)
        mB = jnp.min(cand, axis=1, keepdims=True)
        d_an_subs.append(mB.astype(jnp.float32))
    m_rank2 = jnp.concatenate(d_an_subs, axis=0)

    sq_blk = jnp.sum(x_blk * x_blk, axis=1, keepdims=True)
    d_an = jnp.sqrt(jnp.clip(m_rank2 + sq_blk, 1e-12, None))

    diff = x_blk - p_blk_ref[...] + 1e-6
    d_ap = jnp.sqrt(jnp.sum(diff * diff, axis=1, keepdims=True))
    part = jnp.sum(jnp.maximum(d_ap - d_an + _MARGIN, 0.0),
                   keepdims=True)

    @pl.when(i == 0)
    def _init():
        out_ref[...] = jnp.zeros((1, 1), jnp.float32)
    out_ref[...] += part

    @pl.when(i == _N // _BLK - 1)
    def _finish():
        out_ref[...] = out_ref[...] * (1.0 / _N)


def kernel(inputs, positive):
    grid = (_N // _BLK,)
    out = pl.pallas_call(
        _triphard_block,
        grid=grid,
        in_specs=[
            pl.BlockSpec((_BLK, _D), lambda i: (i, 0)),
            pl.BlockSpec((_N, _D), lambda i: (0, 0)),
        ],
        out_specs=pl.BlockSpec((1, 1), lambda i: (0, 0)),
        out_shape=jax.ShapeDtypeStruct((1, 1), jnp.float32),
        scratch_shapes=[pltpu.VMEM((8, _N), jnp.bfloat16),
                        pltpu.VMEM((_N, _D), jnp.bfloat16)],
    )(positive, inputs)
    return out[0, 0]
